# baseline (speedup 1.0000x reference)
import jax
import jax.numpy as jnp
from jax import lax
from jax.experimental import pallas as pl
from jax.experimental.pallas import tpu as pltpu
from jax.experimental.pallas import tpu_sc as plsc

_THR = 0.5
_EPS = 1e-7
_BIG = 1e9
_BIG_TEST = 1e8

_NW = 32
_ROWS = 128
_D = 5
_MPAD = 1024
_CHUNKS = _MPAD // 16
_KBLK = 8
_NSC = _NW * _ROWS
_TCB = 256


def _body(pred_hbm, dpt_hbm, out_hbm, pred_v, dpt_v, der_v, out_v, fmin_s):
    cid = lax.axis_index("c")
    sid = lax.axis_index("s")
    wid = sid * 2 + cid

    pltpu.sync_copy(pred_hbm.at[wid], pred_v)
    pltpu.sync_copy(dpt_hbm, dpt_v)

    def pre(i, carry):
        d = i // _CHUNKS
        cc = (i % _CHUNKS) * 16
        base = d * (4 * _MPAD) + cc
        cx = dpt_v[pl.ds(base, 16)]
        cy = dpt_v[pl.ds(base + _MPAD, 16)]
        w = dpt_v[pl.ds(base + 2 * _MPAD, 16)]
        h = dpt_v[pl.ds(base + 3 * _MPAD, 16)]
        x1 = cx - w * 0.5
        x2 = cx + w * 0.5
        y1 = cy - h * 0.5
        y2 = cy + h * 0.5
        a2 = (x2 - x1) * (y2 - y1)
        ob = d * _MPAD + cc
        der_v[pl.ds(ob, 16)] = x1
        der_v[pl.ds(_D * _MPAD + ob, 16)] = x2
        der_v[pl.ds(2 * _D * _MPAD + ob, 16)] = y1
        der_v[pl.ds(3 * _D * _MPAD + ob, 16)] = y2
        der_v[pl.ds(4 * _D * _MPAD + ob, 16)] = a2
        return carry

    lax.fori_loop(0, _D * _CHUNKS, pre, 0)

    iota = lax.iota(jnp.int32, 16)
    two_iota = iota.astype(jnp.float32) * 2.0
    bigv = jnp.full((16,), _BIG, jnp.float32)
    shuf_idx = [(iota + s) % 16 for s in (8, 4, 2, 1)]

    def _lane_min(v):
        for idx in shuf_idx:
            v = jnp.minimum(v, v.at[idx].get(mode="promise_in_bounds"))
        return v[0]

    def row(i, out_vec):
        rowv = pred_v[pl.ds(i * 16, 16)]
        cx = rowv[0]
        cy = rowv[1]
        w = rowv[2]
        h = rowv[3]
        x1 = cx - w * 0.5
        x2 = cx + w * 0.5
        y1 = cy - h * 0.5
        y2 = cy + h * 0.5
        a1 = (x2 - x1) * (y2 - y1)

        if True:
            px1 = jnp.full((16,), x1, jnp.float32)
            px2 = jnp.full((16,), x2, jnp.float32)
            py1 = jnp.full((16,), y1, jnp.float32)
            py2 = jnp.full((16,), y2, jnp.float32)
            va1 = jnp.full((16,), a1, jnp.float32)
            sum_s = jnp.float32(0.0)
            cnt_s = jnp.float32(0.0)
            skip0 = jnp.where(a1 > 0.0, _BIG, 0.0)
            for d in range(_D):
                dbase = d * _MPAD
                fmin_s[0] = skip0

                def blk(b, carry):
                    @pl.when(fmin_s[0] > _BIG_TEST)
                    def _():
                        acc = bigv
                        c0 = b * _KBLK
                        jv2 = (jnp.full((16,), (c0 * 32 + 2).astype(
                            jnp.float32), jnp.float32) + two_iota)
                        for k in range(_KBLK):
                            o = dbase + (c0 + k) * 16
                            dx1 = der_v[pl.ds(o, 16)]
                            dx2 = der_v[pl.ds(_D * _MPAD + o, 16)]
                            dy1 = der_v[pl.ds(2 * _D * _MPAD + o, 16)]
                            dy2 = der_v[pl.ds(3 * _D * _MPAD + o, 16)]
                            da2 = der_v[pl.ds(4 * _D * _MPAD + o, 16)]
                            iw = jnp.maximum(
                                jnp.minimum(px2, dx2) - jnp.maximum(px1, dx1),
                                0.0)
                            ih = jnp.maximum(
                                jnp.minimum(py2, dy2) - jnp.maximum(py1, dy1),
                                0.0)
                            inter = iw * ih
                            union = (va1 + da2) - inter + _EPS
                            iou = inter / union
                            comb = jnp.where(iou > _THR, jv2 - iou, bigv)
                            acc = jnp.minimum(acc, comb)
                            if k + 1 < _KBLK:
                                jv2 = jv2 + 32.0
                        fmin_s[0] = _lane_min(acc)
                    return carry

                lax.fori_loop(0, _CHUNKS // _KBLK, blk, 0)
                fmin = fmin_s[0]
                valid = (fmin < _BIG_TEST) & (fmin > 0.0)
                tr = ((fmin - 1.0) * 0.5).astype(jnp.int32)
                iou_val = (tr * 2 + 2).astype(jnp.float32) - fmin
                sum_s = sum_s + jnp.where(valid, iou_val, 0.0)
                cnt_s = cnt_s + jnp.where(valid, 1.0, 0.0)
            rcp = jnp.where(
                cnt_s < 1.5, 1.0,
                jnp.where(cnt_s < 2.5, 0.5,
                          jnp.where(cnt_s < 3.5, jnp.float32(1.0 / 3.0),
                                    jnp.where(cnt_s < 4.5, 0.25, 0.2))))
            mean = jnp.where(cnt_s > 0.0, sum_s * rcp, 0.0)
            out_s = (mean - _THR) * 2.0

        lane = lax.rem(i, 16)
        out_vec = jnp.where(iota == lane, jnp.full((16,), out_s, jnp.float32),
                            out_vec)

        @pl.when(lane == 15)
        def _():
            out_v[pl.ds(i - 15, 16)] = out_vec

        return out_vec

    lax.fori_loop(0, _ROWS, row, jnp.zeros((16,), jnp.float32))
    pltpu.sync_copy(out_v, out_hbm.at[wid])


def _tc_body(pred_ref, dpt_ref, out_ref):
    pr = pred_ref[...]
    cx = pr[:, 0:1]
    cy = pr[:, 1:2]
    w = pr[:, 2:3]
    h = pr[:, 3:4]
    px1 = cx - w * 0.5
    px2 = cx + w * 0.5
    py1 = cy - h * 0.5
    py2 = cy + h * 0.5
    a1 = (px2 - px1) * (py2 - py1)
    key = lax.broadcasted_iota(jnp.int32, (1, _MPAD), 1).astype(
        jnp.float32) * 2.0 + 2.0
    sums = jnp.zeros((_TCB,), jnp.float32)
    cnts = jnp.zeros((_TCB,), jnp.float32)
    for d in range(_D):
        dp = dpt_ref[d]
        dcx = dp[0:1, :]
        dcy = dp[1:2, :]
        dw = dp[2:3, :]
        dh = dp[3:4, :]
        dx1 = dcx - dw * 0.5
        dx2 = dcx + dw * 0.5
        dy1 = dcy - dh * 0.5
        dy2 = dcy + dh * 0.5
        da2 = (dx2 - dx1) * (dy2 - dy1)
        iw = jnp.maximum(jnp.minimum(px2, dx2) - jnp.maximum(px1, dx1), 0.0)
        ih = jnp.maximum(jnp.minimum(py2, dy2) - jnp.maximum(py1, dy1), 0.0)
        inter = iw * ih
        union = (a1 + da2) - inter + _EPS
        iou = inter / union
        comb = jnp.where(iou > _THR, key - iou, _BIG)
        fmin = jnp.min(comb, axis=1)
        valid = (fmin < _BIG_TEST) & (fmin > 0.0)
        tr = ((fmin - 1.0) * 0.5).astype(jnp.int32)
        iou_val = (tr * 2 + 2).astype(jnp.float32) - fmin
        sums = sums + jnp.where(valid, iou_val, 0.0)
        cnts = cnts + jnp.where(valid, 1.0, 0.0)
    mean = jnp.where(cnts > 0.0, sums / jnp.maximum(cnts, 1.0), 0.0)
    out_ref[...] = (mean - _THR) * 2.0


def _tc_run(pred_tc, dpt3):
    nt = pred_tc.shape[0]
    return pl.pallas_call(
        _tc_body,
        grid=(nt // _TCB,),
        in_specs=[
            pl.BlockSpec((_TCB, 8), lambda i: (i, 0)),
            pl.BlockSpec((_D, 4, _MPAD), lambda i: (0, 0, 0)),
        ],
        out_specs=pl.BlockSpec((_TCB,), lambda i: (i,)),
        out_shape=jax.ShapeDtypeStruct((nt,), jnp.float32),
    )(pred_tc, dpt3)


@jax.jit
def _run(pred32, dpt_flat):
    mesh = plsc.VectorSubcoreMesh(core_axis_name="c", subcore_axis_name="s")
    return pl.kernel(
        _body,
        out_type=jax.ShapeDtypeStruct((_NW, _ROWS), jnp.float32),
        mesh=mesh,
        scratch_types=[
            pltpu.VMEM((_ROWS * 16,), jnp.float32),
            pltpu.VMEM((_D * 4 * _MPAD,), jnp.float32),
            pltpu.VMEM((5 * _D * _MPAD,), jnp.float32),
            pltpu.VMEM((_ROWS,), jnp.float32),
            pltpu.SMEM((1,), jnp.float32),
        ],
    )(pred32, dpt_flat)


@jax.jit
def _hybrid(pred, dpt):
    pred_sc = jnp.pad(pred[:_NSC], ((0, 0), (0, 10))).reshape(
        _NW, _ROWS * 16)
    out_sc = _run(pred_sc, dpt.reshape(-1))
    ptc = pred[_NSC:]
    nt = ptc.shape[0]
    ntp = -(-nt // _TCB) * _TCB
    ptc = jnp.pad(ptc, ((0, ntp - nt), (0, 2)))
    out_tc = _tc_run(ptc, dpt)
    return jnp.concatenate([out_sc.reshape(-1), out_tc[:nt]])


def kernel(pred, dropout_preds, dropout_cls_confs):
    del dropout_cls_confs
    n = pred.shape[0]
    dpt = jnp.transpose(dropout_preds[:, :, :4], (0, 2, 1))
    dpt = jnp.pad(dpt, ((0, 0), (0, 0), (0, _MPAD - dpt.shape[2])))
    return _hybrid(pred[:, :6], dpt)[:n]


# speedup vs baseline: 1.0350x; 1.0350x over previous
import jax
import jax.numpy as jnp
from jax import lax
from jax.experimental import pallas as pl
from jax.experimental.pallas import tpu as pltpu
from jax.experimental.pallas import tpu_sc as plsc

_THR = 0.5
_EPS = 1e-7
_BIG = 1e9
_BIG_TEST = 1e8

_NW = 32
_ROWS = 160
_D = 5
_MPAD = 1024
_CHUNKS = _MPAD // 16
_KBLK = 8
_NSC = _NW * _ROWS
_TCB = 256


def _body(pred_hbm, dpt_hbm, out_hbm, pred_v, dpt_v, der_v, out_v, fmin_s):
    cid = lax.axis_index("c")
    sid = lax.axis_index("s")
    wid = sid * 2 + cid

    pltpu.sync_copy(pred_hbm.at[wid], pred_v)
    pltpu.sync_copy(dpt_hbm, dpt_v)

    def pre(i, carry):
        d = i // _CHUNKS
        cc = (i % _CHUNKS) * 16
        base = d * (4 * _MPAD) + cc
        cx = dpt_v[pl.ds(base, 16)]
        cy = dpt_v[pl.ds(base + _MPAD, 16)]
        w = dpt_v[pl.ds(base + 2 * _MPAD, 16)]
        h = dpt_v[pl.ds(base + 3 * _MPAD, 16)]
        x1 = cx - w * 0.5
        x2 = cx + w * 0.5
        y1 = cy - h * 0.5
        y2 = cy + h * 0.5
        a2 = (x2 - x1) * (y2 - y1)
        ob = d * _MPAD + cc
        der_v[pl.ds(ob, 16)] = x1
        der_v[pl.ds(_D * _MPAD + ob, 16)] = x2
        der_v[pl.ds(2 * _D * _MPAD + ob, 16)] = y1
        der_v[pl.ds(3 * _D * _MPAD + ob, 16)] = y2
        der_v[pl.ds(4 * _D * _MPAD + ob, 16)] = a2
        return carry

    lax.fori_loop(0, _D * _CHUNKS, pre, 0)

    iota = lax.iota(jnp.int32, 16)
    two_iota = iota.astype(jnp.float32) * 2.0
    bigv = jnp.full((16,), _BIG, jnp.float32)
    shuf_idx = [(iota + s) % 16 for s in (8, 4, 2, 1)]

    def _lane_min(v):
        for idx in shuf_idx:
            v = jnp.minimum(v, v.at[idx].get(mode="promise_in_bounds"))
        return v[0]

    def row(i, out_vec):
        rowv = pred_v[pl.ds(i * 16, 16)]
        cx = rowv[0]
        cy = rowv[1]
        w = rowv[2]
        h = rowv[3]
        x1 = cx - w * 0.5
        x2 = cx + w * 0.5
        y1 = cy - h * 0.5
        y2 = cy + h * 0.5
        a1 = (x2 - x1) * (y2 - y1)

        if True:
            px1 = jnp.full((16,), x1, jnp.float32)
            px2 = jnp.full((16,), x2, jnp.float32)
            py1 = jnp.full((16,), y1, jnp.float32)
            py2 = jnp.full((16,), y2, jnp.float32)
            va1 = jnp.full((16,), a1, jnp.float32)
            sum_s = jnp.float32(0.0)
            cnt_s = jnp.float32(0.0)
            skip0 = jnp.where(a1 > 0.0, _BIG, 0.0)
            for d in range(_D):
                dbase = d * _MPAD
                fmin_s[0] = skip0

                def blk(b, carry):
                    @pl.when(fmin_s[0] > _BIG_TEST)
                    def _():
                        acc = bigv
                        c0 = b * _KBLK
                        jv2 = (jnp.full((16,), (c0 * 32 + 2).astype(
                            jnp.float32), jnp.float32) + two_iota)
                        for k in range(_KBLK):
                            o = dbase + (c0 + k) * 16
                            dx1 = der_v[pl.ds(o, 16)]
                            dx2 = der_v[pl.ds(_D * _MPAD + o, 16)]
                            dy1 = der_v[pl.ds(2 * _D * _MPAD + o, 16)]
                            dy2 = der_v[pl.ds(3 * _D * _MPAD + o, 16)]
                            da2 = der_v[pl.ds(4 * _D * _MPAD + o, 16)]
                            iw = jnp.maximum(
                                jnp.minimum(px2, dx2) - jnp.maximum(px1, dx1),
                                0.0)
                            ih = jnp.maximum(
                                jnp.minimum(py2, dy2) - jnp.maximum(py1, dy1),
                                0.0)
                            inter = iw * ih
                            union = (va1 + da2) - inter + _EPS
                            iou = inter / union
                            comb = jnp.where(iou > _THR, jv2 - iou, bigv)
                            acc = jnp.minimum(acc, comb)
                            if k + 1 < _KBLK:
                                jv2 = jv2 + 32.0
                        fmin_s[0] = _lane_min(acc)
                    return carry

                lax.fori_loop(0, _CHUNKS // _KBLK, blk, 0)
                fmin = fmin_s[0]
                valid = (fmin < _BIG_TEST) & (fmin > 0.0)
                tr = ((fmin - 1.0) * 0.5).astype(jnp.int32)
                iou_val = (tr * 2 + 2).astype(jnp.float32) - fmin
                sum_s = sum_s + jnp.where(valid, iou_val, 0.0)
                cnt_s = cnt_s + jnp.where(valid, 1.0, 0.0)
            rcp = jnp.where(
                cnt_s < 1.5, 1.0,
                jnp.where(cnt_s < 2.5, 0.5,
                          jnp.where(cnt_s < 3.5, jnp.float32(1.0 / 3.0),
                                    jnp.where(cnt_s < 4.5, 0.25, 0.2))))
            mean = jnp.where(cnt_s > 0.0, sum_s * rcp, 0.0)
            out_s = (mean - _THR) * 2.0

        lane = lax.rem(i, 16)
        out_vec = jnp.where(iota == lane, jnp.full((16,), out_s, jnp.float32),
                            out_vec)

        @pl.when(lane == 15)
        def _():
            out_v[pl.ds(i - 15, 16)] = out_vec

        return out_vec

    lax.fori_loop(0, _ROWS, row, jnp.zeros((16,), jnp.float32))
    pltpu.sync_copy(out_v, out_hbm.at[wid])


def _tc_body(pred_ref, dpt_ref, out_ref):
    pr = pred_ref[...]
    cx = pr[:, 0:1]
    cy = pr[:, 1:2]
    w = pr[:, 2:3]
    h = pr[:, 3:4]
    px1 = cx - w * 0.5
    px2 = cx + w * 0.5
    py1 = cy - h * 0.5
    py2 = cy + h * 0.5
    a1 = (px2 - px1) * (py2 - py1)
    key = lax.broadcasted_iota(jnp.int32, (1, _MPAD), 1).astype(
        jnp.float32) * 2.0 + 2.0
    sums = jnp.zeros((_TCB,), jnp.float32)
    cnts = jnp.zeros((_TCB,), jnp.float32)
    for d in range(_D):
        dp = dpt_ref[d]
        dcx = dp[0:1, :]
        dcy = dp[1:2, :]
        dw = dp[2:3, :]
        dh = dp[3:4, :]
        dx1 = dcx - dw * 0.5
        dx2 = dcx + dw * 0.5
        dy1 = dcy - dh * 0.5
        dy2 = dcy + dh * 0.5
        da2 = (dx2 - dx1) * (dy2 - dy1)
        iw = jnp.maximum(jnp.minimum(px2, dx2) - jnp.maximum(px1, dx1), 0.0)
        ih = jnp.maximum(jnp.minimum(py2, dy2) - jnp.maximum(py1, dy1), 0.0)
        inter = iw * ih
        union = (a1 + da2) - inter + _EPS
        iou = inter / union
        comb = jnp.where(iou > _THR, key - iou, _BIG)
        fmin = jnp.min(comb, axis=1)
        valid = (fmin < _BIG_TEST) & (fmin > 0.0)
        tr = ((fmin - 1.0) * 0.5).astype(jnp.int32)
        iou_val = (tr * 2 + 2).astype(jnp.float32) - fmin
        sums = sums + jnp.where(valid, iou_val, 0.0)
        cnts = cnts + jnp.where(valid, 1.0, 0.0)
    mean = jnp.where(cnts > 0.0, sums / jnp.maximum(cnts, 1.0), 0.0)
    out_ref[...] = (mean - _THR) * 2.0


def _tc_run(pred_tc, dpt3):
    nt = pred_tc.shape[0]
    return pl.pallas_call(
        _tc_body,
        grid=(nt // _TCB,),
        in_specs=[
            pl.BlockSpec((_TCB, 8), lambda i: (i, 0)),
            pl.BlockSpec((_D, 4, _MPAD), lambda i: (0, 0, 0)),
        ],
        out_specs=pl.BlockSpec((_TCB,), lambda i: (i,)),
        out_shape=jax.ShapeDtypeStruct((nt,), jnp.float32),
    )(pred_tc, dpt3)


@jax.jit
def _run(pred32, dpt_flat):
    mesh = plsc.VectorSubcoreMesh(core_axis_name="c", subcore_axis_name="s")
    return pl.kernel(
        _body,
        out_type=jax.ShapeDtypeStruct((_NW, _ROWS), jnp.float32),
        mesh=mesh,
        scratch_types=[
            pltpu.VMEM((_ROWS * 16,), jnp.float32),
            pltpu.VMEM((_D * 4 * _MPAD,), jnp.float32),
            pltpu.VMEM((5 * _D * _MPAD,), jnp.float32),
            pltpu.VMEM((_ROWS,), jnp.float32),
            pltpu.SMEM((1,), jnp.float32),
        ],
    )(pred32, dpt_flat)


@jax.jit
def _hybrid(pred, dpt):
    pred_sc = jnp.pad(pred[:_NSC], ((0, 0), (0, 10))).reshape(
        _NW, _ROWS * 16)
    out_sc = _run(pred_sc, dpt.reshape(-1))
    ptc = pred[_NSC:]
    nt = ptc.shape[0]
    ntp = -(-nt // _TCB) * _TCB
    ptc = jnp.pad(ptc, ((0, ntp - nt), (0, 2)))
    out_tc = _tc_run(ptc, dpt)
    return jnp.concatenate([out_sc.reshape(-1), out_tc[:nt]])


def kernel(pred, dropout_preds, dropout_cls_confs):
    del dropout_cls_confs
    n = pred.shape[0]
    dpt = jnp.transpose(dropout_preds[:, :, :4], (0, 2, 1))
    dpt = jnp.pad(dpt, ((0, 0), (0, 0), (0, _MPAD - dpt.shape[2])))
    return _hybrid(pred[:, :6], dpt)[:n]
